# baseline (device time: 23314 ns/iter reference)
import jax
import jax.numpy as jnp
from jax import lax
from jax.experimental import pallas as pl
from jax.experimental.pallas import tpu as pltpu

N_DEV = 16
M = 512
N = 512
K = 256
ROWS = M // N_DEV
HN = N // 2


def kernel(A, B):
    def body(a_hbm_ref, b_hbm_ref, out_ref, partial_ref, rs_ref, a_ref, b_ref,
             send_l, recv_l, send_r, recv_r, ag_send_l, ag_recv_l,
             ag_send_r, ag_recv_r, ready_sems, copy_sems):
        my = lax.axis_index("i")

        a_copy = pltpu.make_async_copy(a_hbm_ref, a_ref, copy_sems.at[0])
        b_copy = pltpu.make_async_copy(b_hbm_ref, b_ref, copy_sems.at[1])
        a_copy.start()
        b_copy.start()

        for d in range(1, N_DEV):
            tgt = (my + d) % N_DEV
            pl.semaphore_signal(
                ready_sems.at[my], inc=1,
                device_id=(tgt,), device_id_type=pl.DeviceIdType.MESH,
            )
        barrier_sem = pltpu.get_barrier_semaphore()
        pl.semaphore_signal(barrier_sem, inc=1)
        pl.semaphore_wait(barrier_sem, 1)

        a_copy.wait()
        b_copy.wait()
        b = b_ref[...].astype(jnp.bfloat16)

        def rs_send(tgt, d, col0, sems_s, sems_r):
            rdma = pltpu.make_async_remote_copy(
                src_ref=partial_ref.at[pl.ds(tgt * ROWS, ROWS),
                                       pl.ds(col0, HN)],
                dst_ref=rs_ref.at[my, :, pl.ds(col0, HN)],
                send_sem=sems_s.at[d - 1],
                recv_sem=sems_r.at[my],
                device_id=(tgt,),
                device_id_type=pl.DeviceIdType.MESH,
            )
            rdma.start()
            return rdma

        def rs_recv(src, d, col0, sems_s, sems_r):
            return pltpu.make_async_remote_copy(
                src_ref=partial_ref.at[pl.ds(0, ROWS), pl.ds(col0, HN)],
                dst_ref=rs_ref.at[src, :, pl.ds(col0, HN)],
                send_sem=sems_s.at[d - 1],
                recv_sem=sems_r.at[src],
                device_id=(src,),
                device_id_type=pl.DeviceIdType.MESH,
            )

        def ag_send(tgt, d, col0, sems_s, sems_r):
            rdma = pltpu.make_async_remote_copy(
                src_ref=out_ref.at[pl.ds(my * ROWS, ROWS), pl.ds(col0, HN)],
                dst_ref=out_ref.at[pl.ds(my * ROWS, ROWS), pl.ds(col0, HN)],
                send_sem=sems_s.at[d - 1],
                recv_sem=sems_r.at[my],
                device_id=(tgt,),
                device_id_type=pl.DeviceIdType.MESH,
            )
            rdma.start()
            return rdma

        def ag_recv(src, d, col0, sems_s, sems_r):
            return pltpu.make_async_remote_copy(
                src_ref=out_ref.at[pl.ds(0, ROWS), pl.ds(col0, HN)],
                dst_ref=out_ref.at[pl.ds(src * ROWS, ROWS), pl.ds(col0, HN)],
                send_sem=sems_s.at[d - 1],
                recv_sem=sems_r.at[src],
                device_id=(src,),
                device_id_type=pl.DeviceIdType.MESH,
            )

        def gelu(z):
            return 0.5 * z * (
                1.0 + jnp.tanh(0.7978845608 * (z + 0.044715 * z * z * z))
            )

        sends = []
        for d in range(1, N_DEV):
            tgt = (my + d) % N_DEV
            a_s = a_ref[pl.ds(tgt * ROWS, ROWS), :].astype(jnp.bfloat16)
            partial_ref[pl.ds(tgt * ROWS, ROWS), :] = jnp.dot(
                a_s, b, preferred_element_type=jnp.float32
            ).astype(jnp.bfloat16)
            pl.semaphore_wait(ready_sems.at[tgt], 1)
            sends.append(rs_send(tgt, d, 0, send_l, recv_l))
            sends.append(rs_send(tgt, d, HN, send_r, recv_r))

        a_s = a_ref[pl.ds(my * ROWS, ROWS), :].astype(jnp.bfloat16)
        acc = jnp.dot(a_s, b, preferred_element_type=jnp.float32)
        acc_l = acc[:, :HN]
        acc_r = acc[:, HN:]

        for d in range(N_DEV - 1, 0, -1):
            src = (my + d) % N_DEV
            rs_recv(src, d, 0, send_l, recv_l).wait_recv()
            acc_l = acc_l + rs_ref[src, :, :HN].astype(jnp.float32)
        out_ref[pl.ds(my * ROWS, ROWS), pl.ds(0, HN)] = gelu(acc_l).astype(
            jnp.bfloat16
        )
        for d in range(1, N_DEV):
            tgt = (my + d) % N_DEV
            sends.append(ag_send(tgt, d, 0, ag_send_l, ag_recv_l))

        for d in range(N_DEV - 1, 0, -1):
            src = (my + d) % N_DEV
            rs_recv(src, d, HN, send_r, recv_r).wait_recv()
            acc_r = acc_r + rs_ref[src, :, HN:].astype(jnp.float32)
        out_ref[pl.ds(my * ROWS, ROWS), pl.ds(HN, HN)] = gelu(acc_r).astype(
            jnp.bfloat16
        )
        for d in range(1, N_DEV):
            tgt = (my + d) % N_DEV
            sends.append(ag_send(tgt, d, HN, ag_send_r, ag_recv_r))

        for d in range(1, N_DEV):
            src = (my + d) % N_DEV
            ag_recv(src, d, 0, ag_send_l, ag_recv_l).wait_recv()
        for d in range(1, N_DEV):
            src = (my + d) % N_DEV
            ag_recv(src, d, HN, ag_send_r, ag_recv_r).wait_recv()

        for rdma in sends:
            rdma.wait_send()

    out_shape = jax.ShapeDtypeStruct((M, N), jnp.bfloat16)
    return pl.pallas_call(
        body,
        out_shape=out_shape,
        in_specs=[
            pl.BlockSpec(memory_space=pl.ANY),
            pl.BlockSpec(memory_space=pl.ANY),
        ],
        out_specs=pl.BlockSpec(memory_space=pltpu.VMEM),
        scratch_shapes=[
            pltpu.VMEM((M, N), jnp.bfloat16),
            pltpu.VMEM((N_DEV, ROWS, N), jnp.bfloat16),
            pltpu.VMEM((M, K), jnp.float32),
            pltpu.VMEM((K, N), jnp.float32),
            pltpu.SemaphoreType.DMA((N_DEV - 1,)),
            pltpu.SemaphoreType.DMA((N_DEV,)),
            pltpu.SemaphoreType.DMA((N_DEV - 1,)),
            pltpu.SemaphoreType.DMA((N_DEV,)),
            pltpu.SemaphoreType.DMA((N_DEV - 1,)),
            pltpu.SemaphoreType.DMA((N_DEV,)),
            pltpu.SemaphoreType.DMA((N_DEV - 1,)),
            pltpu.SemaphoreType.DMA((N_DEV,)),
            pltpu.SemaphoreType.REGULAR((N_DEV,)),
            pltpu.SemaphoreType.DMA((2,)),
        ],
        compiler_params=pltpu.CompilerParams(collective_id=0),
    )(A, B)


# device time: 21967 ns/iter; 1.0613x vs baseline; 1.0613x over previous
import jax
import jax.numpy as jnp
from jax import lax
from jax.experimental import pallas as pl
from jax.experimental.pallas import tpu as pltpu

N_DEV = 16
M = 512
N = 512
K = 256
ROWS = M // N_DEV


def kernel(A, B):
    def body(a_hbm_ref, b_hbm_ref, out_ref, partial_ref, rs_ref, a_ref, b_ref,
             send_sems, recv_sems, send_sems2, recv_sems2, ready_sems,
             copy_sems):
        my = lax.axis_index("i")

        a_copy = pltpu.make_async_copy(a_hbm_ref, a_ref, copy_sems.at[0])
        b_copy = pltpu.make_async_copy(b_hbm_ref, b_ref, copy_sems.at[1])
        a_copy.start()
        b_copy.start()

        for d in range(1, N_DEV):
            tgt = (my + d) % N_DEV
            pl.semaphore_signal(
                ready_sems.at[my], inc=1,
                device_id=(tgt,), device_id_type=pl.DeviceIdType.MESH,
            )
        barrier_sem = pltpu.get_barrier_semaphore()
        pl.semaphore_signal(barrier_sem, inc=1)
        pl.semaphore_wait(barrier_sem, 1)

        a_copy.wait()
        b_copy.wait()
        b = b_ref[...].astype(jnp.bfloat16)

        sends = []
        for d in range(1, N_DEV):
            tgt = (my + d) % N_DEV
            a_s = a_ref[pl.ds(tgt * ROWS, ROWS), :].astype(jnp.bfloat16)
            partial_ref[pl.ds(tgt * ROWS, ROWS), :] = jnp.dot(
                a_s, b, preferred_element_type=jnp.float32
            ).astype(jnp.bfloat16)
            pl.semaphore_wait(ready_sems.at[tgt], 1)
            rdma = pltpu.make_async_remote_copy(
                src_ref=partial_ref.at[pl.ds(tgt * ROWS, ROWS), :],
                dst_ref=rs_ref.at[my],
                send_sem=send_sems.at[d - 1],
                recv_sem=recv_sems.at[my],
                device_id=(tgt,),
                device_id_type=pl.DeviceIdType.MESH,
            )
            rdma.start()
            sends.append(rdma)

        a_s = a_ref[pl.ds(my * ROWS, ROWS), :].astype(jnp.bfloat16)
        acc = jnp.dot(a_s, b, preferred_element_type=jnp.float32)

        for d in range(N_DEV - 1, 0, -1):
            src = (my + d) % N_DEV
            recv = pltpu.make_async_remote_copy(
                src_ref=partial_ref.at[pl.ds(0, ROWS), :],
                dst_ref=rs_ref.at[src],
                send_sem=send_sems.at[d - 1],
                recv_sem=recv_sems.at[src],
                device_id=(src,),
                device_id_type=pl.DeviceIdType.MESH,
            )
            recv.wait_recv()
            acc = acc + rs_ref[src].astype(jnp.float32)

        z = acc
        g = 0.5 * z * (1.0 + jnp.tanh(0.7978845608 * (z + 0.044715 * z * z * z)))
        out_ref[pl.ds(my * ROWS, ROWS), :] = g.astype(jnp.bfloat16)

        for d in range(1, N_DEV):
            tgt = (my + d) % N_DEV
            rdma = pltpu.make_async_remote_copy(
                src_ref=out_ref.at[pl.ds(my * ROWS, ROWS), :],
                dst_ref=out_ref.at[pl.ds(my * ROWS, ROWS), :],
                send_sem=send_sems2.at[d - 1],
                recv_sem=recv_sems2.at[my],
                device_id=(tgt,),
                device_id_type=pl.DeviceIdType.MESH,
            )
            rdma.start()
            sends.append(rdma)

        for d in range(1, N_DEV):
            src = (my + d) % N_DEV
            recv = pltpu.make_async_remote_copy(
                src_ref=out_ref.at[pl.ds(0, ROWS), :],
                dst_ref=out_ref.at[pl.ds(src * ROWS, ROWS), :],
                send_sem=send_sems2.at[d - 1],
                recv_sem=recv_sems2.at[src],
                device_id=(src,),
                device_id_type=pl.DeviceIdType.MESH,
            )
            recv.wait_recv()

        for rdma in sends:
            rdma.wait_send()

    out_shape = jax.ShapeDtypeStruct((M, N), jnp.bfloat16)
    return pl.pallas_call(
        body,
        out_shape=out_shape,
        in_specs=[
            pl.BlockSpec(memory_space=pl.ANY),
            pl.BlockSpec(memory_space=pl.ANY),
        ],
        out_specs=pl.BlockSpec(memory_space=pltpu.VMEM),
        scratch_shapes=[
            pltpu.VMEM((M, N), jnp.bfloat16),
            pltpu.VMEM((N_DEV, ROWS, N), jnp.bfloat16),
            pltpu.VMEM((M, K), jnp.float32),
            pltpu.VMEM((K, N), jnp.float32),
            pltpu.SemaphoreType.DMA((N_DEV - 1,)),
            pltpu.SemaphoreType.DMA((N_DEV,)),
            pltpu.SemaphoreType.DMA((N_DEV - 1,)),
            pltpu.SemaphoreType.DMA((N_DEV,)),
            pltpu.SemaphoreType.REGULAR((N_DEV,)),
            pltpu.SemaphoreType.DMA((2,)),
        ],
        compiler_params=pltpu.CompilerParams(collective_id=0),
    )(A, B)
